# baseline (speedup 1.0000x reference)
import functools

import jax
import jax.numpy as jnp
from jax import lax
from jax.experimental import pallas as pl
from jax.experimental.pallas import tpu as pltpu
from jax.experimental.pallas import tpu_sc as plsc

NC = 2
NS = 16
NW = NC * NS
LANES = 16
CHUNK = 256
RS_PAD = 48


def _seg_min_kernel(n_rows, n_seg_out, seg_pw, d, dim_size):
    d_vecs = d // LANES
    tail = n_seg_out - (NW - 1) * seg_pw
    mesh = plsc.VectorSubcoreMesh(core_axis_name="c", subcore_axis_name="s")

    @functools.partial(
        pl.kernel,
        mesh=mesh,
        out_type=jax.ShapeDtypeStruct((n_seg_out, d), jnp.float32),
        scratch_types=[
            pltpu.VMEM((RS_PAD,), jnp.int32),
            pltpu.VMEM((2, CHUNK, d), jnp.float32),
            pltpu.VMEM((2, CHUNK), jnp.int32),
            pltpu.VMEM((seg_pw + 8, d), jnp.float32),
            pltpu.SemaphoreType.DMA((2,)),
            pltpu.SemaphoreType.DMA((2,)),
        ],
        compiler_params=pltpu.CompilerParams(needs_layout_passes=False),
    )
    def k(emb_hbm, idx_hbm, rs_hbm, out_hbm, rs_v, emb_v, idx_v, loc_v,
          esem, isem):
        wid = lax.axis_index("s") * NC + lax.axis_index("c")
        pltpu.sync_copy(rs_hbm, rs_v)
        lane = lax.broadcasted_iota(jnp.int32, (LANES,), 0)
        bound_idx = wid + jnp.minimum(lane, 1)
        bounds = plsc.load_gather(rs_v, [bound_idx])
        row_lo = bounds[0]
        row_hi = bounds[1]
        seg_lo = wid * seg_pw

        inf_v = jnp.full((LANES,), jnp.inf, jnp.float32)
        inf_acc = (inf_v,) * d_vecs

        def fill_body(i, _):
            for j in range(d_vecs):
                loc_v[i, pl.ds(j * LANES, LANES)] = inf_v
            return 0

        lax.fori_loop(0, seg_pw + 1, fill_body, 0)

        def flush_to(cur, acc):
            off = cur - seg_lo
            for j in range(d_vecs):
                loc_v[off, pl.ds(j * LANES, LANES)] = acc[j]

        c_lo = row_lo // CHUNK
        c_hi = (row_hi + CHUNK - 1) // CHUNK

        def start_chunk(c, p):
            base = c * CHUNK
            pltpu.async_copy(emb_hbm.at[pl.ds(base, CHUNK)], emb_v.at[p],
                             esem.at[p])
            pltpu.async_copy(idx_hbm.at[pl.ds(base, CHUNK)], idx_v.at[p],
                             isem.at[p])

        def wait_chunk(p):
            pltpu.make_async_copy(emb_hbm.at[pl.ds(0, CHUNK)], emb_v.at[p],
                                  esem.at[p]).wait()
            pltpu.make_async_copy(idx_hbm.at[pl.ds(0, CHUNK)], idx_v.at[p],
                                  isem.at[p]).wait()

        @pl.when(c_lo < c_hi)
        def _():
            start_chunk(c_lo, 0)

        def chunk_body(c, carry):
            p = lax.rem(c - c_lo, 2)

            @pl.when(c + 1 < c_hi)
            def _():
                start_chunk(c + 1, 1 - p)

            wait_chunk(p)
            base = c * CHUNK
            r_lo = jnp.maximum(row_lo, base) - base
            r_hi = jnp.minimum(row_hi, base + CHUNK) - base

            def make_group_body(masked):
                def group_body(g, carry):
                    gbase = g * LANES
                    ivec = idx_v[p, pl.ds(gbase, LANES)]
                    s0 = ivec[0]
                    fast = s0 == ivec[LANES - 1]
                    if masked:
                        full = jnp.logical_and(gbase >= r_lo,
                                               gbase + LANES <= r_hi)
                        fast = jnp.logical_and(full, fast)

                    def fast_fn(op):
                        cur, acc = op

                        def do_flush(a):
                            flush_to(cur, a)
                            return inf_acc

                        acc = lax.cond(s0 != cur, do_flush, lambda a: a, acc)
                        new = []
                        for j in range(d_vecs):
                            sl = pl.ds(j * LANES, LANES)
                            m = [emb_v[p, gbase + l, sl] for l in range(LANES)]
                            while len(m) > 1:
                                m = [jnp.minimum(m[i], m[i + 1])
                                     for i in range(0, len(m) - 1, 2)] + (
                                         [m[-1]] if len(m) % 2 else [])
                            new.append(jnp.minimum(acc[j], m[0]))
                        return s0, tuple(new)

                    def slow_fn(op):
                        cur, acc = op
                        for l in range(LANES):
                            r = gbase + l
                            s = ivec[l]
                            change = s != cur
                            if masked:
                                ok = jnp.logical_and(r >= r_lo, r < r_hi)
                                change = jnp.logical_and(ok, change)
                            store_off = jnp.where(change, cur - seg_lo, seg_pw)
                            new = []
                            for j in range(d_vecs):
                                sl = pl.ds(j * LANES, LANES)
                                loc_v[store_off, sl] = acc[j]
                                v = emb_v[p, r, sl]
                                if masked:
                                    v = jnp.where(ok, v, inf_v)
                                a = jnp.where(change, inf_v, acc[j])
                                new.append(jnp.minimum(a, v))
                            acc = tuple(new)
                            cur = jnp.where(change, s, cur)
                        return cur, acc

                    return lax.cond(fast, fast_fn, slow_fn, carry)

                return group_body

            interior = jnp.logical_and(r_lo == 0, r_hi == CHUNK)

            def interior_fn(op):
                return lax.fori_loop(0, CHUNK // LANES, make_group_body(False),
                                     op, unroll=2)

            def boundary_fn(op):
                g_lo = r_lo // LANES
                g_hi = (r_hi + LANES - 1) // LANES
                return lax.fori_loop(g_lo, g_hi, make_group_body(True), op)

            return lax.cond(interior, interior_fn, boundary_fn, carry)

        carry0 = (jnp.int32(seg_lo), inf_acc)
        cur, acc = lax.fori_loop(c_lo, c_hi, chunk_body, carry0)
        flush_to(cur, acc)

        @pl.when(wid < NW - 1)
        def _():
            pltpu.sync_copy(loc_v.at[pl.ds(0, seg_pw)],
                            out_hbm.at[pl.ds(seg_lo, seg_pw)])

        @pl.when(wid == NW - 1)
        def _():
            pltpu.sync_copy(loc_v.at[pl.ds(0, tail)],
                            out_hbm.at[pl.ds((NW - 1) * seg_pw, tail)])

    return k


def kernel(embeddings, idx, dim_size):
    n, d = embeddings.shape
    assert n % CHUNK == 0
    try:
        dim_size = int(dim_size)
    except (jax.errors.ConcretizationTypeError, TypeError):
        dim_size = 10000
    seg_pw = -(-dim_size // (NW * 8)) * 8
    tail = dim_size - (NW - 1) * seg_pw
    assert 0 < tail <= seg_pw and tail % 8 == 0
    idx32 = idx.astype(jnp.int32)
    bounds = jnp.arange(0, NW * seg_pw + 1, seg_pw, dtype=jnp.int32)
    row_starts = jnp.sum(
        (idx32[:, None] < bounds[None, :]).astype(jnp.int32), axis=0,
        dtype=jnp.int32)
    row_starts = jnp.pad(row_starts, (0, RS_PAD - NW - 1), constant_values=n)
    k = _seg_min_kernel(n, dim_size, seg_pw, d, dim_size)
    return k(embeddings, idx32, row_starts)


# speedup vs baseline: 1.0706x; 1.0706x over previous
import functools

import jax
import jax.numpy as jnp
from jax import lax
from jax.experimental import pallas as pl
from jax.experimental.pallas import tpu as pltpu
from jax.experimental.pallas import tpu_sc as plsc

NC = 2
NS = 16
NW = NC * NS
LANES = 16
CHUNK = 256
RS_PAD = 48


def _seg_min_kernel(n_rows, n_seg_out, seg_pw, d, dim_size):
    d_vecs = d // LANES
    tail = n_seg_out - (NW - 1) * seg_pw
    mesh = plsc.VectorSubcoreMesh(core_axis_name="c", subcore_axis_name="s")

    @functools.partial(
        pl.kernel,
        mesh=mesh,
        out_type=jax.ShapeDtypeStruct((n_seg_out, d), jnp.float32),
        scratch_types=[
            pltpu.VMEM((RS_PAD,), jnp.int32),
            pltpu.VMEM((2, CHUNK, d), jnp.float32),
            pltpu.VMEM((2, CHUNK), jnp.int32),
            pltpu.VMEM((seg_pw + 8, d), jnp.float32),
            pltpu.SemaphoreType.DMA((2,)),
            pltpu.SemaphoreType.DMA((2,)),
        ],
        compiler_params=pltpu.CompilerParams(needs_layout_passes=False),
    )
    def k(emb_hbm, idx_hbm, rs_hbm, out_hbm, rs_v, emb_v, idx_v, loc_v,
          esem, isem):
        wid = lax.axis_index("s") * NC + lax.axis_index("c")
        pltpu.sync_copy(rs_hbm, rs_v)
        lane = lax.broadcasted_iota(jnp.int32, (LANES,), 0)
        bound_idx = wid + jnp.minimum(lane, 1)
        bounds = plsc.load_gather(rs_v, [bound_idx])
        row_lo = bounds[0]
        row_hi = bounds[1]
        seg_lo = wid * seg_pw

        inf_v = jnp.full((LANES,), jnp.inf, jnp.float32)
        inf_acc = (inf_v,) * d_vecs

        def fill_body(i, _):
            for j in range(d_vecs):
                loc_v[i, pl.ds(j * LANES, LANES)] = inf_v
            return 0

        lax.fori_loop(0, seg_pw + 1, fill_body, 0)

        def flush_to(cur, acc):
            off = cur - seg_lo
            for j in range(d_vecs):
                loc_v[off, pl.ds(j * LANES, LANES)] = acc[j]

        c_lo = row_lo // CHUNK
        c_hi = (row_hi + CHUNK - 1) // CHUNK

        def start_chunk(c, p):
            base = c * CHUNK
            pltpu.async_copy(emb_hbm.at[pl.ds(base, CHUNK)], emb_v.at[p],
                             esem.at[p])
            pltpu.async_copy(idx_hbm.at[pl.ds(base, CHUNK)], idx_v.at[p],
                             isem.at[p])

        def wait_chunk(p):
            pltpu.make_async_copy(emb_hbm.at[pl.ds(0, CHUNK)], emb_v.at[p],
                                  esem.at[p]).wait()
            pltpu.make_async_copy(idx_hbm.at[pl.ds(0, CHUNK)], idx_v.at[p],
                                  isem.at[p]).wait()

        @pl.when(c_lo < c_hi)
        def _():
            start_chunk(c_lo, 0)

        def chunk_body(c, carry):
            p = lax.rem(c - c_lo, 2)

            @pl.when(c + 1 < c_hi)
            def _():
                start_chunk(c + 1, 1 - p)

            wait_chunk(p)
            base = c * CHUNK
            r_lo = jnp.maximum(row_lo, base) - base
            r_hi = jnp.minimum(row_hi, base + CHUNK) - base

            def make_group_body(masked):
                def group_body(g, carry):
                    gbase = g * LANES
                    ivec = idx_v[p, pl.ds(gbase, LANES)]
                    s0 = ivec[0]
                    fast = s0 == ivec[LANES - 1]
                    if masked:
                        full = jnp.logical_and(gbase >= r_lo,
                                               gbase + LANES <= r_hi)
                        fast = jnp.logical_and(full, fast)

                    def fast_fn(op):
                        cur, acc = op

                        def do_flush(a):
                            flush_to(cur, a)
                            return inf_acc

                        acc = lax.cond(s0 != cur, do_flush, lambda a: a, acc)
                        new = []
                        for j in range(d_vecs):
                            sl = pl.ds(j * LANES, LANES)
                            m = [emb_v[p, gbase + l, sl] for l in range(LANES)]
                            while len(m) > 1:
                                m = [jnp.minimum(m[i], m[i + 1])
                                     for i in range(0, len(m) - 1, 2)] + (
                                         [m[-1]] if len(m) % 2 else [])
                            new.append(jnp.minimum(acc[j], m[0]))
                        return s0, tuple(new)

                    def slow_fn(op):
                        cur, acc = op
                        for l in range(LANES):
                            r = gbase + l
                            s = ivec[l]
                            change = s != cur
                            if masked:
                                ok = jnp.logical_and(r >= r_lo, r < r_hi)
                                change = jnp.logical_and(ok, change)
                            store_off = jnp.where(change, cur - seg_lo, seg_pw)
                            new = []
                            for j in range(d_vecs):
                                sl = pl.ds(j * LANES, LANES)
                                loc_v[store_off, sl] = acc[j]
                                v = emb_v[p, r, sl]
                                if masked:
                                    v = jnp.where(ok, v, inf_v)
                                a = jnp.where(change, inf_v, acc[j])
                                new.append(jnp.minimum(a, v))
                            acc = tuple(new)
                            cur = jnp.where(change, s, cur)
                        return cur, acc

                    return lax.cond(fast, fast_fn, slow_fn, carry)

                return group_body

            interior = jnp.logical_and(r_lo == 0, r_hi == CHUNK)

            def interior_fn(op):
                return lax.fori_loop(0, CHUNK // LANES, make_group_body(False),
                                     op)

            def boundary_fn(op):
                g_lo = r_lo // LANES
                g_hi = (r_hi + LANES - 1) // LANES
                return lax.fori_loop(g_lo, g_hi, make_group_body(True), op)

            return lax.cond(interior, interior_fn, boundary_fn, carry)

        carry0 = (jnp.int32(seg_lo), inf_acc)
        cur, acc = lax.fori_loop(c_lo, c_hi, chunk_body, carry0)
        flush_to(cur, acc)

        @pl.when(wid < NW - 1)
        def _():
            pltpu.sync_copy(loc_v.at[pl.ds(0, seg_pw)],
                            out_hbm.at[pl.ds(seg_lo, seg_pw)])

        @pl.when(wid == NW - 1)
        def _():
            pltpu.sync_copy(loc_v.at[pl.ds(0, tail)],
                            out_hbm.at[pl.ds((NW - 1) * seg_pw, tail)])

    return k


def kernel(embeddings, idx, dim_size):
    n, d = embeddings.shape
    assert n % CHUNK == 0
    try:
        dim_size = int(dim_size)
    except (jax.errors.ConcretizationTypeError, TypeError):
        dim_size = 10000
    seg_pw = -(-dim_size // (NW * 8)) * 8
    tail = dim_size - (NW - 1) * seg_pw
    assert 0 < tail <= seg_pw and tail % 8 == 0
    idx32 = idx.astype(jnp.int32)
    bounds = jnp.arange(0, NW * seg_pw + 1, seg_pw, dtype=jnp.int32)
    row_starts = jnp.sum(
        (idx32[:, None] < bounds[None, :]).astype(jnp.int32), axis=0,
        dtype=jnp.int32)
    row_starts = jnp.pad(row_starts, (0, RS_PAD - NW - 1), constant_values=n)
    k = _seg_min_kernel(n, dim_size, seg_pw, d, dim_size)
    return k(embeddings, idx32, row_starts)


# speedup vs baseline: 1.0859x; 1.0142x over previous
import functools

import jax
import jax.numpy as jnp
from jax import lax
from jax.experimental import pallas as pl
from jax.experimental.pallas import tpu as pltpu
from jax.experimental.pallas import tpu_sc as plsc

NC = 2
NS = 16
NW = NC * NS
LANES = 16
CHUNK = 256
RS_PAD = 48


def _seg_min_kernel(n_rows, n_seg_out, seg_pw, d, dim_size):
    d_vecs = d // LANES
    tail = n_seg_out - (NW - 1) * seg_pw
    mesh = plsc.VectorSubcoreMesh(core_axis_name="c", subcore_axis_name="s")

    @functools.partial(
        pl.kernel,
        mesh=mesh,
        out_type=jax.ShapeDtypeStruct((n_seg_out, d), jnp.float32),
        scratch_types=[
            pltpu.VMEM((RS_PAD,), jnp.int32),
            pltpu.VMEM((2, CHUNK, d), jnp.float32),
            pltpu.VMEM((2, CHUNK), jnp.int32),
            pltpu.VMEM((seg_pw + 8, d), jnp.float32),
            pltpu.SemaphoreType.DMA((2,)),
            pltpu.SemaphoreType.DMA((2,)),
        ],
        compiler_params=pltpu.CompilerParams(needs_layout_passes=False),
    )
    def k(emb_hbm, idx_hbm, rs_hbm, out_hbm, rs_v, emb_v, idx_v, loc_v,
          esem, isem):
        wid = lax.axis_index("s") * NC + lax.axis_index("c")
        pltpu.sync_copy(rs_hbm, rs_v)
        lane = lax.broadcasted_iota(jnp.int32, (LANES,), 0)
        bound_idx = wid + jnp.minimum(lane, 1)
        bounds = plsc.load_gather(rs_v, [bound_idx])
        row_lo = bounds[0]
        row_hi = bounds[1]
        seg_lo = wid * seg_pw

        inf_v = jnp.full((LANES,), jnp.inf, jnp.float32)
        inf_acc = (inf_v,) * d_vecs

        c_lo = row_lo // CHUNK
        c_hi = (row_hi + CHUNK - 1) // CHUNK

        def start_chunk(c, p):
            base = c * CHUNK
            pltpu.async_copy(emb_hbm.at[pl.ds(base, CHUNK)], emb_v.at[p],
                             esem.at[p])
            pltpu.async_copy(idx_hbm.at[pl.ds(base, CHUNK)], idx_v.at[p],
                             isem.at[p])

        @pl.when(c_lo < c_hi)
        def _():
            start_chunk(c_lo, 0)

        def fill_body(i, _):
            for j in range(d_vecs):
                loc_v[i, pl.ds(j * LANES, LANES)] = inf_v
            return 0

        lax.fori_loop(0, seg_pw + 1, fill_body, 0)

        def flush_to(cur, acc):
            off = cur - seg_lo
            for j in range(d_vecs):
                loc_v[off, pl.ds(j * LANES, LANES)] = acc[j]

        def wait_chunk(p):
            pltpu.make_async_copy(emb_hbm.at[pl.ds(0, CHUNK)], emb_v.at[p],
                                  esem.at[p]).wait()
            pltpu.make_async_copy(idx_hbm.at[pl.ds(0, CHUNK)], idx_v.at[p],
                                  isem.at[p]).wait()

        def chunk_body(c, carry):
            p = lax.rem(c - c_lo, 2)

            @pl.when(c + 1 < c_hi)
            def _():
                start_chunk(c + 1, 1 - p)

            wait_chunk(p)
            base = c * CHUNK
            r_lo = jnp.maximum(row_lo, base) - base
            r_hi = jnp.minimum(row_hi, base + CHUNK) - base

            def make_group_body(masked):
                def group_body(g, carry):
                    gbase = g * LANES
                    ivec = idx_v[p, pl.ds(gbase, LANES)]
                    s0 = ivec[0]
                    fast = s0 == ivec[LANES - 1]
                    if masked:
                        full = jnp.logical_and(gbase >= r_lo,
                                               gbase + LANES <= r_hi)
                        fast = jnp.logical_and(full, fast)

                    def fast_fn(op):
                        cur, acc = op

                        def do_flush(a):
                            flush_to(cur, a)
                            return inf_acc

                        acc = lax.cond(s0 != cur, do_flush, lambda a: a, acc)
                        new = []
                        for j in range(d_vecs):
                            sl = pl.ds(j * LANES, LANES)
                            m = [emb_v[p, gbase + l, sl] for l in range(LANES)]
                            while len(m) > 1:
                                m = [jnp.minimum(m[i], m[i + 1])
                                     for i in range(0, len(m) - 1, 2)] + (
                                         [m[-1]] if len(m) % 2 else [])
                            new.append(jnp.minimum(acc[j], m[0]))
                        return s0, tuple(new)

                    def slow_fn(op):
                        cur, acc = op
                        for l in range(LANES):
                            r = gbase + l
                            s = ivec[l]
                            change = s != cur
                            if masked:
                                ok = jnp.logical_and(r >= r_lo, r < r_hi)
                                change = jnp.logical_and(ok, change)
                            store_off = jnp.where(change, cur - seg_lo, seg_pw)
                            new = []
                            for j in range(d_vecs):
                                sl = pl.ds(j * LANES, LANES)
                                loc_v[store_off, sl] = acc[j]
                                v = emb_v[p, r, sl]
                                if masked:
                                    v = jnp.where(ok, v, inf_v)
                                a = jnp.where(change, inf_v, acc[j])
                                new.append(jnp.minimum(a, v))
                            acc = tuple(new)
                            cur = jnp.where(change, s, cur)
                        return cur, acc

                    return lax.cond(fast, fast_fn, slow_fn, carry)

                return group_body

            interior = jnp.logical_and(r_lo == 0, r_hi == CHUNK)

            def interior_fn(op):
                return lax.fori_loop(0, CHUNK // LANES, make_group_body(False),
                                     op)

            def boundary_fn(op):
                g_lo = r_lo // LANES
                g_hi = (r_hi + LANES - 1) // LANES
                return lax.fori_loop(g_lo, g_hi, make_group_body(True), op)

            return lax.cond(interior, interior_fn, boundary_fn, carry)

        carry0 = (jnp.int32(seg_lo), inf_acc)
        cur, acc = lax.fori_loop(c_lo, c_hi, chunk_body, carry0)
        flush_to(cur, acc)

        @pl.when(wid < NW - 1)
        def _():
            pltpu.sync_copy(loc_v.at[pl.ds(0, seg_pw)],
                            out_hbm.at[pl.ds(seg_lo, seg_pw)])

        @pl.when(wid == NW - 1)
        def _():
            pltpu.sync_copy(loc_v.at[pl.ds(0, tail)],
                            out_hbm.at[pl.ds((NW - 1) * seg_pw, tail)])

    return k


def kernel(embeddings, idx, dim_size):
    n, d = embeddings.shape
    assert n % CHUNK == 0
    try:
        dim_size = int(dim_size)
    except (jax.errors.ConcretizationTypeError, TypeError):
        dim_size = 10000
    seg_pw = -(-dim_size // (NW * 8)) * 8
    tail = dim_size - (NW - 1) * seg_pw
    assert 0 < tail <= seg_pw and tail % 8 == 0
    idx32 = idx.astype(jnp.int32)
    bounds = jnp.arange(0, NW * seg_pw + 1, seg_pw, dtype=jnp.int32)
    row_starts = jnp.sum(
        (idx32[:, None] < bounds[None, :]).astype(jnp.int32), axis=0,
        dtype=jnp.int32)
    row_starts = jnp.pad(row_starts, (0, RS_PAD - NW - 1), constant_values=n)
    k = _seg_min_kernel(n, dim_size, seg_pw, d, dim_size)
    return k(embeddings, idx32, row_starts)
